# baseline (speedup 1.0000x reference)
import functools

import jax
import jax.numpy as jnp
from jax import lax
from jax.experimental import pallas as pl
from jax.experimental.pallas import tpu as pltpu
from jax.experimental.pallas import tpu_sc as plsc


def _sc_embed_gather(embed, idx):
    t = idx.shape[0]
    d = embed.shape[1]
    info = plsc.get_sparse_core_info()
    nw = info.num_cores * info.num_subcores
    bpw = t // nw
    mesh = plsc.VectorSubcoreMesh(core_axis_name="c", subcore_axis_name="s")

    @functools.partial(
        pl.kernel,
        mesh=mesh,
        out_type=jax.ShapeDtypeStruct((t, d), jnp.float32),
        scratch_types=[
            pltpu.VMEM((bpw,), jnp.int32),
            pltpu.VMEM((bpw, d), jnp.float32),
            pltpu.SemaphoreType.DMA,
        ],
    )
    def gather_kernel(table_hbm, idx_hbm, out_hbm, idx_v, rows_v, sem):
        wid = lax.axis_index("s") * info.num_cores + lax.axis_index("c")
        base = wid * bpw
        pltpu.sync_copy(idx_hbm.at[pl.ds(base, bpw)], idx_v)
        pltpu.async_copy(table_hbm.at[idx_v], rows_v, sem).wait()
        pltpu.sync_copy(rows_v, out_hbm.at[pl.ds(base, bpw)])

    return gather_kernel(embed, idx)


def _gate_mlp_ln_body(h_ref, wg_ref, bg_ref, w1_ref, b1_ref, w2_ref, b2_ref,
                      gamma_ref, beta_ref, out_ref, idx_ref):
    h = h_ref[...]
    e = wg_ref.shape[1]
    g = jnp.dot(h, wg_ref[...], preferred_element_type=jnp.float32) + bg_ref[...]
    m = jnp.max(g, axis=-1, keepdims=True)
    ex = jnp.exp(g - m)
    probs = ex / jnp.sum(ex, axis=-1, keepdims=True)
    eidx = lax.broadcasted_iota(jnp.int32, probs.shape, 1)
    v0 = jnp.max(probs, axis=-1, keepdims=True)
    i0 = jnp.min(jnp.where(probs == v0, eidx, e), axis=-1, keepdims=True)
    p1 = jnp.where(eidx == i0, -jnp.inf, probs)
    v1 = jnp.max(p1, axis=-1, keepdims=True)
    i1 = jnp.min(jnp.where(p1 == v1, eidx, e), axis=-1, keepdims=True)
    idx_ref[...] = jnp.concatenate([i0, i1], axis=-1)

    hb = h.astype(jnp.bfloat16)
    u = jnp.dot(hb, w1_ref[...].astype(jnp.bfloat16),
                preferred_element_type=jnp.float32) + b1_ref[...]
    u = jnp.maximum(u, 0.0)
    yo = jnp.dot(u.astype(jnp.bfloat16), w2_ref[...].astype(jnp.bfloat16),
                 preferred_element_type=jnp.float32) + b2_ref[...]
    y = yo * v0
    mu = jnp.mean(y, axis=-1, keepdims=True)
    var = jnp.mean((y - mu) ** 2, axis=-1, keepdims=True)
    out = (y - mu) * lax.rsqrt(var + 1e-5) * gamma_ref[...] + beta_ref[...]
    out_ref[...] = out.astype(jnp.bfloat16)


def _gate_mlp_ln(h, wg, bg, w1, b1, w2, b2, gamma, beta):
    t, d = h.shape
    e = wg.shape[1]
    dh = w1.shape[1]
    tb = 256
    grid = (t // tb,)
    res = pl.BlockSpec(memory_space=pltpu.VMEM)
    return pl.pallas_call(
        _gate_mlp_ln_body,
        grid=grid,
        in_specs=[
            pl.BlockSpec((tb, d), lambda i: (i, 0)),
            res, res, res, res, res, res, res, res,
        ],
        out_specs=[
            pl.BlockSpec((tb, d), lambda i: (i, 0)),
            pl.BlockSpec((tb, 2), lambda i: (i, 0)),
        ],
        out_shape=[
            jax.ShapeDtypeStruct((t, d), jnp.bfloat16),
            jax.ShapeDtypeStruct((t, 2), jnp.int32),
        ],
    )(h, wg, bg.reshape(1, e), w1, b1.reshape(1, dh), w2, b2.reshape(1, d),
      gamma.reshape(1, d), beta.reshape(1, d))


def _head_body(act_ref, wh_ref, bh_ref, out_ref):
    m = pl.program_id(1)
    tb = out_ref.shape[0]
    wh = wh_ref[...].astype(jnp.bfloat16)
    a = act_ref[pl.ds(m * tb, tb), :]
    out_ref[...] = jnp.dot(a, wh,
                           preferred_element_type=jnp.float32) + bh_ref[...]


def _head(act, wh, bh):
    t, d = act.shape
    v = wh.shape[1]
    vb = 1280
    tb = t // 2
    grid = (v // vb, 2)
    return pl.pallas_call(
        _head_body,
        grid=grid,
        in_specs=[
            pl.BlockSpec(memory_space=pltpu.VMEM),
            pl.BlockSpec((d, vb), lambda j, m: (0, j)),
            pl.BlockSpec((1, vb), lambda j, m: (0, j)),
        ],
        out_specs=pl.BlockSpec((tb, vb), lambda j, m: (m, j)),
        out_shape=jax.ShapeDtypeStruct((t, v), jnp.float32),
    )(act, wh, bh.reshape(1, v))


def kernel(x, embed, Wg, bg, W1, b1, W2, b2, gamma, beta, Wh, bh):
    b, t = x.shape
    idx = x.reshape(b * t).astype(jnp.int32)
    h = _sc_embed_gather(embed, idx)
    act, topk_idx = _gate_mlp_ln(h, Wg, bg, W1, b1, W2, b2, gamma, beta)
    logits = _head(act, Wh, bh)
    return logits, topk_idx.reshape(b, t, 2)


# speedup vs baseline: 1.2379x; 1.2379x over previous
import functools

import jax
import jax.numpy as jnp
from jax import lax
from jax.experimental import pallas as pl
from jax.experimental.pallas import tpu as pltpu
from jax.experimental.pallas import tpu_sc as plsc


def _sc_embed_gather(embed, idx):
    t = idx.shape[0]
    d = embed.shape[1]
    info = plsc.get_sparse_core_info()
    nw = info.num_cores * info.num_subcores
    bpw = t // nw
    mesh = plsc.VectorSubcoreMesh(core_axis_name="c", subcore_axis_name="s")

    @functools.partial(
        pl.kernel,
        mesh=mesh,
        out_type=jax.ShapeDtypeStruct((t, d), jnp.float32),
        scratch_types=[
            pltpu.VMEM((bpw,), jnp.int32),
            pltpu.VMEM((bpw, d), jnp.float32),
            pltpu.SemaphoreType.DMA,
        ],
    )
    def gather_kernel(table_hbm, idx_hbm, out_hbm, idx_v, rows_v, sem):
        wid = lax.axis_index("s") * info.num_cores + lax.axis_index("c")
        base = wid * bpw
        pltpu.sync_copy(idx_hbm.at[pl.ds(base, bpw)], idx_v)
        pltpu.async_copy(table_hbm.at[idx_v], rows_v, sem).wait()
        pltpu.sync_copy(rows_v, out_hbm.at[pl.ds(base, bpw)])

    return gather_kernel(embed, idx)


def _gate_mlp_ln_body(h_ref, wg_ref, bg_ref, w1_ref, b1_ref, w2_ref, b2_ref,
                      gamma_ref, beta_ref, out_ref, idx_ref):
    h = h_ref[...]
    e = wg_ref.shape[1]
    g = jnp.dot(h, wg_ref[...], preferred_element_type=jnp.float32) + bg_ref[...]
    m = jnp.max(g, axis=-1, keepdims=True)
    ex = jnp.exp(g - m)
    probs = ex / jnp.sum(ex, axis=-1, keepdims=True)
    eidx = lax.broadcasted_iota(jnp.int32, probs.shape, 1)
    v0 = jnp.max(probs, axis=-1, keepdims=True)
    i0 = jnp.min(jnp.where(probs == v0, eidx, e), axis=-1, keepdims=True)
    p1 = jnp.where(eidx == i0, -jnp.inf, probs)
    v1 = jnp.max(p1, axis=-1, keepdims=True)
    i1 = jnp.min(jnp.where(p1 == v1, eidx, e), axis=-1, keepdims=True)
    idx_ref[...] = jnp.concatenate([i0, i1], axis=-1)

    hb = h.astype(jnp.bfloat16)
    u = jnp.dot(hb, w1_ref[...].astype(jnp.bfloat16),
                preferred_element_type=jnp.float32) + b1_ref[...]
    u = jnp.maximum(u, 0.0)
    yo = jnp.dot(u.astype(jnp.bfloat16), w2_ref[...].astype(jnp.bfloat16),
                 preferred_element_type=jnp.float32) + b2_ref[...]
    y = yo * v0
    mu = jnp.mean(y, axis=-1, keepdims=True)
    var = jnp.mean((y - mu) ** 2, axis=-1, keepdims=True)
    out = (y - mu) * lax.rsqrt(var + 1e-5) * gamma_ref[...] + beta_ref[...]
    out_ref[...] = out.astype(jnp.bfloat16)


def _gate_mlp_ln(h, wg, bg, w1, b1, w2, b2, gamma, beta):
    t, d = h.shape
    e = wg.shape[1]
    dh = w1.shape[1]
    tb = 256
    grid = (t // tb,)
    res = pl.BlockSpec(memory_space=pltpu.VMEM)
    return pl.pallas_call(
        _gate_mlp_ln_body,
        grid=grid,
        in_specs=[
            pl.BlockSpec((tb, d), lambda i: (i, 0)),
            res, res, res, res, res, res, res, res,
        ],
        out_specs=[
            pl.BlockSpec((tb, d), lambda i: (i, 0)),
            pl.BlockSpec((tb, 2), lambda i: (i, 0)),
        ],
        out_shape=[
            jax.ShapeDtypeStruct((t, d), jnp.bfloat16),
            jax.ShapeDtypeStruct((t, 2), jnp.int32),
        ],
    )(h, wg, bg.reshape(1, e), w1, b1.reshape(1, dh), w2, b2.reshape(1, d),
      gamma.reshape(1, d), beta.reshape(1, d))


def _head_body(act_ref, wh_ref, bh_ref, out_ref):
    wh = wh_ref[...].astype(jnp.bfloat16)
    out_ref[...] = jnp.dot(act_ref[...], wh,
                           preferred_element_type=jnp.float32) + bh_ref[...]


def _head(act, wh, bh):
    t, d = act.shape
    v = wh.shape[1]
    vb = 1024
    grid = (pl.cdiv(v, vb),)
    return pl.pallas_call(
        _head_body,
        grid=grid,
        in_specs=[
            pl.BlockSpec(memory_space=pltpu.VMEM),
            pl.BlockSpec((d, vb), lambda j: (0, j)),
            pl.BlockSpec((1, vb), lambda j: (0, j)),
        ],
        out_specs=pl.BlockSpec((t, vb), lambda j: (0, j)),
        out_shape=jax.ShapeDtypeStruct((t, v), jnp.float32),
    )(act, wh, bh.reshape(1, v))


def kernel(x, embed, Wg, bg, W1, b1, W2, b2, gamma, beta, Wh, bh):
    b, t = x.shape
    idx = x.reshape(b * t).astype(jnp.int32)
    h = _sc_embed_gather(embed, idx)
    act, topk_idx = _gate_mlp_ln(h, Wg, bg, W1, b1, W2, b2, gamma, beta)
    logits = _head(act, Wh, bh)
    return logits, topk_idx.reshape(b, t, 2)


# speedup vs baseline: 1.2581x; 1.0162x over previous
import functools

import jax
import jax.numpy as jnp
from jax import lax
from jax.experimental import pallas as pl
from jax.experimental.pallas import tpu as pltpu
from jax.experimental.pallas import tpu_sc as plsc


def _sc_embed_gather(embed, idx):
    t = idx.shape[0]
    d = embed.shape[1]
    info = plsc.get_sparse_core_info()
    nw = info.num_cores * info.num_subcores
    bpw = t // nw
    mesh = plsc.VectorSubcoreMesh(core_axis_name="c", subcore_axis_name="s")

    @functools.partial(
        pl.kernel,
        mesh=mesh,
        out_type=jax.ShapeDtypeStruct((t, d), jnp.float32),
        scratch_types=[
            pltpu.VMEM((bpw,), jnp.int32),
            pltpu.VMEM((bpw, d), jnp.float32),
            pltpu.SemaphoreType.DMA,
        ],
    )
    def gather_kernel(table_hbm, idx_hbm, out_hbm, idx_v, rows_v, sem):
        wid = lax.axis_index("s") * info.num_cores + lax.axis_index("c")
        base = wid * bpw
        pltpu.sync_copy(idx_hbm.at[pl.ds(base, bpw)], idx_v)
        pltpu.async_copy(table_hbm.at[idx_v], rows_v, sem).wait()
        pltpu.sync_copy(rows_v, out_hbm.at[pl.ds(base, bpw)])

    return gather_kernel(embed, idx)


def _gate_mlp_ln_body(h_ref, wg_ref, bg_ref, w1_ref, b1_ref, w2_ref, b2_ref,
                      gamma_ref, beta_ref, out_ref, idx_ref):
    h = h_ref[...]
    e = wg_ref.shape[1]
    g = jnp.dot(h, wg_ref[...], preferred_element_type=jnp.float32) + bg_ref[...]
    m = jnp.max(g, axis=-1, keepdims=True)
    ex = jnp.exp(g - m)
    probs = ex / jnp.sum(ex, axis=-1, keepdims=True)
    eidx = lax.broadcasted_iota(jnp.int32, probs.shape, 1)
    v0 = jnp.max(probs, axis=-1, keepdims=True)
    i0 = jnp.min(jnp.where(probs == v0, eidx, e), axis=-1, keepdims=True)
    p1 = jnp.where(eidx == i0, -jnp.inf, probs)
    v1 = jnp.max(p1, axis=-1, keepdims=True)
    i1 = jnp.min(jnp.where(p1 == v1, eidx, e), axis=-1, keepdims=True)
    idx_ref[...] = jnp.concatenate([i0, i1], axis=-1)

    hb = h.astype(jnp.bfloat16)
    u = jnp.dot(hb, w1_ref[...].astype(jnp.bfloat16),
                preferred_element_type=jnp.float32) + b1_ref[...]
    u = jnp.maximum(u, 0.0)
    yo = jnp.dot(u.astype(jnp.bfloat16), w2_ref[...].astype(jnp.bfloat16),
                 preferred_element_type=jnp.float32) + b2_ref[...]
    y = yo * v0
    mu = jnp.mean(y, axis=-1, keepdims=True)
    var = jnp.mean((y - mu) ** 2, axis=-1, keepdims=True)
    out = (y - mu) * lax.rsqrt(var + 1e-5) * gamma_ref[...] + beta_ref[...]
    out_ref[...] = out.astype(jnp.bfloat16)


def _gate_mlp_ln(h, wg, bg, w1, b1, w2, b2, gamma, beta):
    t, d = h.shape
    e = wg.shape[1]
    dh = w1.shape[1]
    tb = 512
    grid = (t // tb,)
    res = pl.BlockSpec(memory_space=pltpu.VMEM)
    return pl.pallas_call(
        _gate_mlp_ln_body,
        grid=grid,
        in_specs=[
            pl.BlockSpec((tb, d), lambda i: (i, 0)),
            res, res, res, res, res, res, res, res,
        ],
        out_specs=[
            pl.BlockSpec((tb, d), lambda i: (i, 0)),
            pl.BlockSpec((tb, 2), lambda i: (i, 0)),
        ],
        out_shape=[
            jax.ShapeDtypeStruct((t, d), jnp.bfloat16),
            jax.ShapeDtypeStruct((t, 2), jnp.int32),
        ],
    )(h, wg, bg.reshape(1, e), w1, b1.reshape(1, dh), w2, b2.reshape(1, d),
      gamma.reshape(1, d), beta.reshape(1, d))


def _head_body(act_ref, wh_ref, bh_ref, out_ref):
    wh = wh_ref[...].astype(jnp.bfloat16)
    out_ref[...] = jnp.dot(act_ref[...], wh,
                           preferred_element_type=jnp.float32) + bh_ref[...]


def _head(act, wh, bh):
    t, d = act.shape
    v = wh.shape[1]
    vb = 1280
    grid = (v // vb,)
    return pl.pallas_call(
        _head_body,
        grid=grid,
        in_specs=[
            pl.BlockSpec(memory_space=pltpu.VMEM),
            pl.BlockSpec((d, vb), lambda j: (0, j)),
            pl.BlockSpec((1, vb), lambda j: (0, j)),
        ],
        out_specs=pl.BlockSpec((t, vb), lambda j: (0, j)),
        out_shape=jax.ShapeDtypeStruct((t, v), jnp.float32),
    )(act, wh, bh.reshape(1, v))


def kernel(x, embed, Wg, bg, W1, b1, W2, b2, gamma, beta, Wh, bh):
    b, t = x.shape
    idx = x.reshape(b * t).astype(jnp.int32)
    h = _sc_embed_gather(embed, idx)
    act, topk_idx = _gate_mlp_ln(h, Wg, bg, W1, b1, W2, b2, gamma, beta)
    logits = _head(act, Wh, bh)
    return logits, topk_idx.reshape(b, t, 2)


# speedup vs baseline: 1.2615x; 1.0028x over previous
import functools

import jax
import jax.numpy as jnp
from jax import lax
from jax.experimental import pallas as pl
from jax.experimental.pallas import tpu as pltpu
from jax.experimental.pallas import tpu_sc as plsc


def _sc_embed_gather(embed, idx):
    t = idx.shape[0]
    d = embed.shape[1]
    info = plsc.get_sparse_core_info()
    nw = info.num_cores * info.num_subcores
    bpw = t // nw
    mesh = plsc.VectorSubcoreMesh(core_axis_name="c", subcore_axis_name="s")

    @functools.partial(
        pl.kernel,
        mesh=mesh,
        out_type=jax.ShapeDtypeStruct((t, d), jnp.float32),
        scratch_types=[
            pltpu.VMEM((bpw,), jnp.int32),
            pltpu.VMEM((bpw, d), jnp.float32),
            pltpu.SemaphoreType.DMA,
        ],
    )
    def gather_kernel(table_hbm, idx_hbm, out_hbm, idx_v, rows_v, sem):
        wid = lax.axis_index("s") * info.num_cores + lax.axis_index("c")
        base = wid * bpw
        pltpu.sync_copy(idx_hbm.at[pl.ds(base, bpw)], idx_v)
        pltpu.async_copy(table_hbm.at[idx_v], rows_v, sem).wait()
        pltpu.sync_copy(rows_v, out_hbm.at[pl.ds(base, bpw)])

    return gather_kernel(embed, idx)


def _gate_mlp_ln_body(h_ref, wg_ref, bg_ref, w1_ref, b1_ref, w2_ref, b2_ref,
                      gamma_ref, beta_ref, out_ref, idx_ref):
    h = h_ref[...]
    e = wg_ref.shape[1]
    g = jnp.dot(h, wg_ref[...], preferred_element_type=jnp.float32) + bg_ref[...]
    m = jnp.max(g, axis=-1, keepdims=True)
    ex = jnp.exp(g - m)
    probs = ex / jnp.sum(ex, axis=-1, keepdims=True)
    eidx = lax.broadcasted_iota(jnp.int32, probs.shape, 1)
    v0 = jnp.max(probs, axis=-1, keepdims=True)
    i0 = jnp.min(jnp.where(probs == v0, eidx, e), axis=-1, keepdims=True)
    p1 = jnp.where(eidx == i0, -jnp.inf, probs)
    v1 = jnp.max(p1, axis=-1, keepdims=True)
    i1 = jnp.min(jnp.where(p1 == v1, eidx, e), axis=-1, keepdims=True)
    idx_ref[...] = jnp.concatenate([i0, i1], axis=-1)

    hb = h.astype(jnp.bfloat16)
    u = jnp.dot(hb, w1_ref[...].astype(jnp.bfloat16),
                preferred_element_type=jnp.float32) + b1_ref[...]
    u = jnp.maximum(u, 0.0)
    yo = jnp.dot(u.astype(jnp.bfloat16), w2_ref[...].astype(jnp.bfloat16),
                 preferred_element_type=jnp.float32) + b2_ref[...]
    y = yo * v0
    mu = jnp.mean(y, axis=-1, keepdims=True)
    var = jnp.mean((y - mu) ** 2, axis=-1, keepdims=True)
    out = (y - mu) * lax.rsqrt(var + 1e-5) * gamma_ref[...] + beta_ref[...]
    out_ref[...] = out


def _gate_mlp_ln(h, wg, bg, w1, b1, w2, b2, gamma, beta):
    t, d = h.shape
    e = wg.shape[1]
    dh = w1.shape[1]
    tb = 512
    grid = (t // tb,)
    res = pl.BlockSpec(memory_space=pltpu.VMEM)
    return pl.pallas_call(
        _gate_mlp_ln_body,
        grid=grid,
        in_specs=[
            pl.BlockSpec((tb, d), lambda i: (i, 0)),
            res, res, res, res, res, res, res, res,
        ],
        out_specs=[
            pl.BlockSpec((tb, d), lambda i: (i, 0)),
            pl.BlockSpec((tb, 2), lambda i: (i, 0)),
        ],
        out_shape=[
            jax.ShapeDtypeStruct((t, d), jnp.float32),
            jax.ShapeDtypeStruct((t, 2), jnp.int32),
        ],
    )(h, wg, bg.reshape(1, e), w1, b1.reshape(1, dh), w2, b2.reshape(1, d),
      gamma.reshape(1, d), beta.reshape(1, d))


def _head_body(act_ref, wh_ref, bh_ref, out_ref):
    out_ref[...] = jnp.dot(act_ref[...], wh_ref[...],
                           preferred_element_type=jnp.float32) + bh_ref[...]


def _head(act, wh, bh):
    t, d = act.shape
    v = wh.shape[1]
    vb = 1280
    grid = (v // vb,)
    return pl.pallas_call(
        _head_body,
        grid=grid,
        in_specs=[
            pl.BlockSpec(memory_space=pltpu.VMEM),
            pl.BlockSpec((d, vb), lambda j: (0, j)),
            pl.BlockSpec((1, vb), lambda j: (0, j)),
        ],
        out_specs=pl.BlockSpec((t, vb), lambda j: (0, j)),
        out_shape=jax.ShapeDtypeStruct((t, v), jnp.float32),
    )(act, wh, bh.reshape(1, v))


def kernel(x, embed, Wg, bg, W1, b1, W2, b2, gamma, beta, Wh, bh):
    b, t = x.shape
    idx = x.reshape(b * t).astype(jnp.int32)
    h = _sc_embed_gather(embed, idx)
    act, topk_idx = _gate_mlp_ln(h, Wg, bg, W1, b1, W2, b2, gamma, beta)
    logits = _head(act, Wh, bh)
    return logits, topk_idx.reshape(b, t, 2)
